# baseline (speedup 1.0000x reference)
import functools

import jax
import jax.numpy as jnp
import numpy as np
from jax import lax
from jax.experimental import pallas as pl
from jax.experimental.pallas import tpu as pltpu
from jax.experimental.pallas import tpu_sc as plsc

_NC = 2
_NS = 16
_NW = _NC * _NS
_LANE = 16


def _make_pool(B, L, H, CH):
    rows_per_w = B // _NW
    n_chunks = rows_per_w // CH
    W = H // 2
    n_grp = W // _LANE
    inv_l = 1.0 / L

    mesh = plsc.VectorSubcoreMesh(
        core_axis_name="c", subcore_axis_name="s",
        num_cores=_NC, num_subcores=_NS)

    @functools.partial(
        pl.kernel,
        out_type=jax.ShapeDtypeStruct((B, H), jnp.float32),
        mesh=mesh,
        compiler_params=pltpu.CompilerParams(use_tc_tiling_on_sc=False),
        scratch_types=[
            pltpu.VMEM((CH * L,), jnp.int32),
            pltpu.VMEM((4, L, W), jnp.int32),
            pltpu.VMEM((CH, H), jnp.float32),
            pltpu.SemaphoreType.DMA,
            pltpu.SemaphoreType.DMA,
            pltpu.SemaphoreType.DMA,
            pltpu.SemaphoreType.DMA,
        ],
    )
    def pool(text_hbm, emb_hbm, out_hbm, idx_v, rows_v, ostage_v,
             sem0, sem1, sem2, sem3):
        wid = lax.axis_index("s") * _NC + lax.axis_index("c")
        base = wid * rows_per_w
        sems = (sem0, sem1, sem2, sem3)

        def start(r, slot):
            pltpu.async_copy(emb_hbm.at[idx_v.at[pl.ds(r * L, L)]],
                             rows_v.at[slot], sems[slot])

        def finish(r, slot):
            pltpu.make_async_copy(emb_hbm.at[idx_v.at[pl.ds(r * L, L)]],
                                  rows_v.at[slot], sems[slot]).wait()

        def reduce_row(slot, r_out):
            def body(t, acc):
                new = []
                los, his = [], []
                for g in range(n_grp):
                    w = rows_v[slot, t, pl.ds(g * _LANE, _LANE)]
                    los.append(acc[g]
                               + lax.bitcast_convert_type(w << 16,
                                                          jnp.float32))
                    his.append(acc[n_grp + g]
                               + lax.bitcast_convert_type(
                                   w & jnp.int32(-65536), jnp.float32))
                return tuple(los + his)
            acc = lax.fori_loop(
                0, L, body,
                tuple(jnp.zeros((_LANE,), jnp.float32)
                      for _ in range(2 * n_grp)),
                unroll=8)
            for j in range(2 * n_grp):
                ostage_v[r_out, pl.ds(j * _LANE, _LANE)] = acc[j] * inv_l

        def chunk_body(c, carry):
            row0 = base + c * CH
            pltpu.sync_copy(text_hbm.at[pl.ds(row0 * L, CH * L)], idx_v)
            for k in range(3):
                start(k, k)

            def quad_body(q, carry2):
                r0 = 4 * q
                for k in range(4):
                    r = r0 + k

                    @pl.when(r + 3 < CH)
                    def _(r=r, k=k):
                        start(r + 3, (k + 3) % 4)

                    finish(r, k)
                    reduce_row(k, r)
                return carry2

            lax.fori_loop(0, CH // 4, quad_body, 0)
            pltpu.sync_copy(ostage_v, out_hbm.at[pl.ds(row0, CH)])
            return carry

        lax.fori_loop(0, n_chunks, chunk_body, 0)

    return pool


def _make_mlp(B, H, F1, F2, BLK):
    def body(x_ref, w1_ref, b1_ref, w2_ref, b2_ref, o_ref):
        x = x_ref[...]
        h = jnp.dot(x, w1_ref[...], preferred_element_type=jnp.float32)
        h = jnp.maximum(h + b1_ref[...], 0.0)
        o = jnp.dot(h, w2_ref[...], preferred_element_type=jnp.float32)
        o_ref[...] = o + b2_ref[...]

    return pl.pallas_call(
        body,
        grid=(B // BLK,),
        in_specs=[
            pl.BlockSpec((BLK, H), lambda i: (i, 0)),
            pl.BlockSpec((H, F1), lambda i: (0, 0)),
            pl.BlockSpec((1, F1), lambda i: (0, 0)),
            pl.BlockSpec((F1, F2), lambda i: (0, 0)),
            pl.BlockSpec((1, F2), lambda i: (0, 0)),
        ],
        out_specs=pl.BlockSpec((BLK, F2), lambda i: (i, 0)),
        out_shape=jax.ShapeDtypeStruct((B, F2), jnp.float32),
    )


def _col_perm(H):
    perm = []
    for g in range(H // 32):
        perm += list(range(32 * g, 32 * g + 32, 2))
        perm += list(range(32 * g + 1, 32 * g + 32, 2))
    return np.array(perm)


def kernel(text, text_lengths, emb, W1, b1, W2, b2):
    del text_lengths
    B, L = text.shape
    V, H = emb.shape
    F1 = W1.shape[1]
    F2 = W2.shape[1]
    text = text.astype(jnp.int32).reshape(B * L)
    Hh = H // 2
    er = jax.lax.bitcast_convert_type(emb, jnp.uint32).reshape(V // 2, 2 * H)
    lo = jnp.concatenate([er[:, 0:Hh], er[:, H:H + Hh]], axis=1)
    hi = jnp.concatenate([er[:, Hh:H], er[:, H + Hh:2 * H]], axis=1)
    lo = lo + jnp.uint32(0x8000)
    hi = hi + jnp.uint32(0x8000)
    packed = jax.lax.bitcast_convert_type(
        (lo >> 16) | (hi & jnp.uint32(0xFFFF0000)), jnp.int32)
    packed = jax.lax.optimization_barrier(packed)
    packed = packed.reshape(V, Hh)
    pooled = _make_pool(B, L, H, CH=64)(text, packed)
    mlp = _make_mlp(B, H, F1, F2, BLK=2048)
    return mlp(pooled, W1, b1.reshape(1, F1), W2, b2.reshape(1, F2))


# speedup vs baseline: 1.2708x; 1.2708x over previous
import functools

import jax
import jax.numpy as jnp
import numpy as np
from jax import lax
from jax.experimental import pallas as pl
from jax.experimental.pallas import tpu as pltpu
from jax.experimental.pallas import tpu_sc as plsc

_NC = 2
_NS = 16
_NW = _NC * _NS
_LANE = 16


def _make_pool(B, L, H, CH):
    rows_per_w = B // _NW
    n_chunks = rows_per_w // CH
    W = H // 2
    n_grp = W // _LANE
    inv_l = 1.0 / L

    mesh = plsc.VectorSubcoreMesh(
        core_axis_name="c", subcore_axis_name="s",
        num_cores=_NC, num_subcores=_NS)

    @functools.partial(
        pl.kernel,
        out_type=jax.ShapeDtypeStruct((B, H), jnp.float32),
        mesh=mesh,
        compiler_params=pltpu.CompilerParams(use_tc_tiling_on_sc=False),
        scratch_types=[
            pltpu.VMEM((CH * L,), jnp.int32),
            pltpu.VMEM((4, L, W), jnp.int32),
            pltpu.VMEM((CH, H), jnp.float32),
            pltpu.SemaphoreType.DMA,
            pltpu.SemaphoreType.DMA,
            pltpu.SemaphoreType.DMA,
            pltpu.SemaphoreType.DMA,
        ],
    )
    def pool(text_hbm, emb_hbm, out_hbm, idx_v, rows_v, ostage_v,
             sem0, sem1, sem2, sem3):
        wid = lax.axis_index("s") * _NC + lax.axis_index("c")
        base = wid * rows_per_w
        sems = (sem0, sem1, sem2, sem3)

        def start(r, slot):
            pltpu.async_copy(emb_hbm.at[idx_v.at[pl.ds(r * L, L)]],
                             rows_v.at[slot], sems[slot])

        def finish(r, slot):
            pltpu.make_async_copy(emb_hbm.at[idx_v.at[pl.ds(r * L, L)]],
                                  rows_v.at[slot], sems[slot]).wait()

        def reduce_row(slot, r_out):
            def body(t, acc):
                new = []
                los, his = [], []
                for g in range(n_grp):
                    w = rows_v[slot, t, pl.ds(g * _LANE, _LANE)]
                    los.append(acc[g]
                               + lax.bitcast_convert_type(w << 16,
                                                          jnp.float32))
                    his.append(acc[n_grp + g]
                               + lax.bitcast_convert_type(
                                   w & jnp.int32(-65536), jnp.float32))
                return tuple(los + his)
            acc = lax.fori_loop(
                0, L, body,
                tuple(jnp.zeros((_LANE,), jnp.float32)
                      for _ in range(2 * n_grp)),
                unroll=8)
            for j in range(2 * n_grp):
                ostage_v[r_out, pl.ds(j * _LANE, _LANE)] = acc[j] * inv_l

        def chunk_body(c, carry):
            row0 = base + c * CH
            pltpu.sync_copy(text_hbm.at[pl.ds(row0 * L, CH * L)], idx_v)
            for k in range(3):
                start(k, k)

            def quad_body(q, carry2):
                r0 = 4 * q
                for k in range(4):
                    r = r0 + k

                    @pl.when(r + 3 < CH)
                    def _(r=r, k=k):
                        start(r + 3, (k + 3) % 4)

                    finish(r, k)
                    reduce_row(k, r)
                return carry2

            lax.fori_loop(0, CH // 4, quad_body, 0)
            pltpu.sync_copy(ostage_v, out_hbm.at[pl.ds(row0, CH)])
            return carry

        lax.fori_loop(0, n_chunks, chunk_body, 0)

    return pool


def _make_mlp(B, H, F1, F2, BLK):
    def body(x_ref, w1_ref, b1_ref, w2_ref, b2_ref, o_ref):
        x = x_ref[...]
        h = jnp.dot(x, w1_ref[...], preferred_element_type=jnp.float32)
        h = jnp.maximum(h + b1_ref[...], 0.0)
        o = jnp.dot(h, w2_ref[...], preferred_element_type=jnp.float32)
        o_ref[...] = o + b2_ref[...]

    return pl.pallas_call(
        body,
        grid=(B // BLK,),
        in_specs=[
            pl.BlockSpec((BLK, H), lambda i: (i, 0)),
            pl.BlockSpec((H, F1), lambda i: (0, 0)),
            pl.BlockSpec((1, F1), lambda i: (0, 0)),
            pl.BlockSpec((F1, F2), lambda i: (0, 0)),
            pl.BlockSpec((1, F2), lambda i: (0, 0)),
        ],
        out_specs=pl.BlockSpec((BLK, F2), lambda i: (i, 0)),
        out_shape=jax.ShapeDtypeStruct((B, F2), jnp.float32),
    )


def _make_pack(V, H, RB):
    rows_per_w = V // _NW
    n_chunks = rows_per_w // RB
    Hh = H // 2
    n_grp = Hh // _LANE

    mesh = plsc.VectorSubcoreMesh(
        core_axis_name="c", subcore_axis_name="s",
        num_cores=_NC, num_subcores=_NS)

    @functools.partial(
        pl.kernel,
        out_type=jax.ShapeDtypeStruct((V, Hh), jnp.int32),
        mesh=mesh,
        compiler_params=pltpu.CompilerParams(use_tc_tiling_on_sc=False),
        scratch_types=[
            pltpu.VMEM((2, RB, H), jnp.float32),
            pltpu.VMEM((2, RB, Hh), jnp.int32),
            pltpu.SemaphoreType.DMA,
            pltpu.SemaphoreType.DMA,
        ],
    )
    def packk(emb_hbm, out_hbm, fin_v, pout_v, semi, semo):
        wid = lax.axis_index("s") * _NC + lax.axis_index("c")
        base = wid * rows_per_w

        def start_in(c, s):
            pltpu.async_copy(emb_hbm.at[pl.ds(base + c * RB, RB)],
                             fin_v.at[s], semi)

        def wait_in(c, s):
            pltpu.make_async_copy(emb_hbm.at[pl.ds(base + c * RB, RB)],
                                  fin_v.at[s], semi).wait()

        def start_out(c, s):
            pltpu.async_copy(pout_v.at[s],
                             out_hbm.at[pl.ds(base + c * RB, RB)], semo)

        def wait_out(c, s):
            pltpu.make_async_copy(pout_v.at[s],
                                  out_hbm.at[pl.ds(base + c * RB, RB)],
                                  semo).wait()

        start_in(0, 0)

        def chunk_body(c, carry):
            s = c % 2

            @pl.when(c + 1 < n_chunks)
            def _():
                start_in(c + 1, 1 - s)

            wait_in(c, s)

            @pl.when(c >= 2)
            def _():
                wait_out(c - 2, s)

            def row_body(r, carry2):
                for g in range(n_grp):
                    a = lax.bitcast_convert_type(
                        fin_v[s, r, pl.ds(g * _LANE, _LANE)], jnp.int32)
                    b = lax.bitcast_convert_type(
                        fin_v[s, r, pl.ds(Hh + g * _LANE, _LANE)], jnp.int32)
                    a = a + 32768
                    b = b + 32768
                    w = (lax.shift_right_logical(a, 16)
                         | (b & jnp.int32(-65536)))
                    pout_v[s, r, pl.ds(g * _LANE, _LANE)] = w
                return carry2

            lax.fori_loop(0, RB, row_body, 0, unroll=4)
            start_out(c, s)
            return carry

        lax.fori_loop(0, n_chunks, chunk_body, 0)
        wait_out(n_chunks - 2, n_chunks % 2)
        wait_out(n_chunks - 1, (n_chunks - 1) % 2)

    return packk


def kernel(text, text_lengths, emb, W1, b1, W2, b2):
    del text_lengths
    B, L = text.shape
    V, H = emb.shape
    F1 = W1.shape[1]
    F2 = W2.shape[1]
    text = text.astype(jnp.int32).reshape(B * L)
    packed = _make_pack(V, H, RB=125)(emb)
    pooled = _make_pool(B, L, H, CH=64)(text, packed)
    mlp = _make_mlp(B, H, F1, F2, BLK=2048)
    return mlp(pooled, W1, b1.reshape(1, F1), W2, b2.reshape(1, F2))
